# baseline (speedup 1.0000x reference)
import jax
import jax.numpy as jnp
from jax.experimental import pallas as pl
from jax.experimental.pallas import tpu as pltpu

_SCALE = 1.0507009873554805
_ALPHA = 1.6732632423543772
_DIAMETER = 2

_DNT = (((1,), (1,)), ((), ()))


def _selu(x):
    em1 = jnp.exp(jnp.minimum(x, 0.0)) - 1.0
    return _SCALE * jnp.where(x > 0, x, _ALPHA * em1)


def _mmt(x, w):
    return jax.lax.dot_general(x, w, _DNT,
                               preferred_element_type=jnp.float32)


def _mpnn_kernel(h_ref, ef_ref, Wmsg_ref, we_ref, bm_ref,
                 Wih_ref, Whh_ref, bih_ref, bhh_ref,
                 Wr1_ref, br1_ref, Wr2_ref, br2_ref, Wp_ref, bp_ref,
                 out_ref):
    w_e = we_ref[...]
    N = h_ref.shape[1]
    D = h_ref.shape[2]
    sa = _SCALE * _ALPHA
    Ws = Wmsg_ref[:, :D]
    Wt = Wmsg_ref[:, D:2 * D]

    for g in range(h_ref.shape[0]):
      h = h_ref[g]
      ef = ef_ref[g]
      E = ef[:, :, None] * w_e[0][None, None, :]

      for _ in range(_DIAMETER):
        Xs = _mmt(h, Ws)
        Xt = _mmt(h, Wt) + bm_ref[...]
        TS = 8
        pos = jnp.zeros((N, D), jnp.float32)
        esum = jnp.zeros((N, D), jnp.float32)
        for c in range(N // TS):
            sl = slice(c * TS, (c + 1) * TS)
            blk = Xs[sl][:, None, :] + Xt[None, :, :] + E[sl]
            pos = pos + jnp.sum(jnp.maximum(blk, 0.0), axis=0)
            esum = esum + jnp.sum(jnp.exp(jnp.minimum(blk, 0.0)), axis=0)
        agg = _SCALE * pos + sa * esum - (sa * N)
        gi = _mmt(agg, Wih_ref[...]) + bih_ref[...]
        gh = _mmt(h, Whh_ref[...]) + bhh_ref[...]
        i_r, i_z, i_n = gi[:, :D], gi[:, D:2 * D], gi[:, 2 * D:]
        h_r, h_z, h_n = gh[:, :D], gh[:, D:2 * D], gh[:, 2 * D:]
        r = jax.nn.sigmoid(i_r + h_r)
        z = jax.nn.sigmoid(i_z + h_z)
        n = jnp.tanh(i_n + r * h_n)
        h = (1.0 - z) * n + z * h

      ns = jnp.sum(h, axis=0, keepdims=True)
      r1 = _selu(_mmt(ns, Wr1_ref[...]) + br1_ref[...])
      r2 = _selu(_mmt(r1, Wr2_ref[...]) + br2_ref[...])
      out_ref[g] = _mmt(r2, Wp_ref[...]) + bp_ref[...]


def kernel(node_features, edge_features, adjacency_matrix,
           W_msg, b_msg, W_ih, W_hh, b_ih, b_hh,
           W_r1, b_r1, W_r2, b_r2, W_p, b_p):
    B, N, D = node_features.shape
    A = W_p.shape[0]

    w_e = W_msg[:, 2 * D].reshape(1, D)

    full = lambda shape: pl.BlockSpec(shape, lambda b: (0,) * len(shape))
    out = pl.pallas_call(
        _mpnn_kernel,
        grid=(1,),
        in_specs=[
            pl.BlockSpec((B, N, D), lambda b: (0, 0, 0)),
            pl.BlockSpec((B, N, N), lambda b: (0, 0, 0)),
            full((D, 2 * D + 1)), full((1, D)), full((1, D)),
            full((3 * D, D)), full((3 * D, D)), full((1, 3 * D)),
            full((1, 3 * D)),
            full((D, D)), full((1, D)), full((D, D)), full((1, D)),
            full((A, D)), full((1, A)),
        ],
        out_specs=pl.BlockSpec((B, 1, A), lambda b: (0, 0, 0)),
        out_shape=jax.ShapeDtypeStruct((B, 1, A), jnp.float32),
        compiler_params=pltpu.CompilerParams(
            dimension_semantics=("parallel",)),
    )(node_features, edge_features, W_msg, w_e, b_msg.reshape(1, D),
      W_ih, W_hh, b_ih.reshape(1, 3 * D), b_hh.reshape(1, 3 * D),
      W_r1, b_r1.reshape(1, D), W_r2, b_r2.reshape(1, D),
      W_p, b_p.reshape(1, A))
    return out.reshape(B, A)


# speedup vs baseline: 1.0312x; 1.0312x over previous
import jax
import jax.numpy as jnp
from jax.experimental import pallas as pl
from jax.experimental.pallas import tpu as pltpu

_SCALE = 1.0507009873554805
_ALPHA = 1.6732632423543772
_DIAMETER = 2

_DNT = (((1,), (1,)), ((), ()))


def _selu(x):
    em1 = jnp.exp(jnp.minimum(x, 0.0)) - 1.0
    return _SCALE * jnp.where(x > 0, x, _ALPHA * em1)


def _mmt(x, w):
    return jax.lax.dot_general(x, w, _DNT,
                               preferred_element_type=jnp.float32)


def _mpnn_kernel(h_ref, ef_ref, Wmsg_ref, we_ref, bm_ref,
                 Wih_ref, Whh_ref, bih_ref, bhh_ref,
                 Wr1_ref, br1_ref, Wr2_ref, br2_ref, Wp_ref, bp_ref,
                 out_ref):
    h = h_ref[0]
    ef = ef_ref[0]
    w_e = we_ref[...]
    N = h.shape[0]
    D = h.shape[1]
    sa = _SCALE * _ALPHA
    Ws = Wmsg_ref[:, :D]
    Wt = Wmsg_ref[:, D:2 * D]

    ef_r = ef.astype(jnp.bfloat16).astype(jnp.float32)
    we_r = w_e.astype(jnp.bfloat16).astype(jnp.float32)
    E = ef_r[:, :, None] * we_r[0][None, None, :]

    for _ in range(_DIAMETER):
        Xs = _mmt(h, Ws)
        Xt = _mmt(h, Wt) + bm_ref[...]
        TS = 8
        pos = jnp.zeros((N, D), jnp.float32)
        esum = jnp.zeros((N, D), jnp.float32)
        for c in range(N // TS):
            sl = slice(c * TS, (c + 1) * TS)
            blk = Xs[sl][:, None, :] + Xt[None, :, :] + E[sl]
            pos = pos + jnp.sum(jnp.maximum(blk, 0.0), axis=0)
            esum = esum + jnp.sum(jnp.exp(jnp.minimum(blk, 0.0)), axis=0)
        agg = _SCALE * pos + sa * esum - (sa * N)
        gi = _mmt(agg, Wih_ref[...]) + bih_ref[...]
        gh = _mmt(h, Whh_ref[...]) + bhh_ref[...]
        i_r, i_z, i_n = gi[:, :D], gi[:, D:2 * D], gi[:, 2 * D:]
        h_r, h_z, h_n = gh[:, :D], gh[:, D:2 * D], gh[:, 2 * D:]
        r = jax.nn.sigmoid(i_r + h_r)
        z = jax.nn.sigmoid(i_z + h_z)
        n = jnp.tanh(i_n + r * h_n)
        h = (1.0 - z) * n + z * h

    ns = jnp.sum(h, axis=0, keepdims=True)
    r1 = _selu(_mmt(ns, Wr1_ref[...]) + br1_ref[...])
    r2 = _selu(_mmt(r1, Wr2_ref[...]) + br2_ref[...])
    out_ref[0] = _mmt(r2, Wp_ref[...]) + bp_ref[...]


def kernel(node_features, edge_features, adjacency_matrix,
           W_msg, b_msg, W_ih, W_hh, b_ih, b_hh,
           W_r1, b_r1, W_r2, b_r2, W_p, b_p):
    B, N, D = node_features.shape
    A = W_p.shape[0]

    w_e = W_msg[:, 2 * D].reshape(1, D)

    full = lambda shape: pl.BlockSpec(shape, lambda b: (0,) * len(shape))
    out = pl.pallas_call(
        _mpnn_kernel,
        grid=(B,),
        in_specs=[
            pl.BlockSpec((1, N, D), lambda b: (b, 0, 0)),
            pl.BlockSpec((1, N, N), lambda b: (b, 0, 0)),
            full((D, 2 * D + 1)), full((1, D)), full((1, D)),
            full((3 * D, D)), full((3 * D, D)), full((1, 3 * D)),
            full((1, 3 * D)),
            full((D, D)), full((1, D)), full((D, D)), full((1, D)),
            full((A, D)), full((1, A)),
        ],
        out_specs=pl.BlockSpec((1, 1, A), lambda b: (b, 0, 0)),
        out_shape=jax.ShapeDtypeStruct((B, 1, A), jnp.float32),
        compiler_params=pltpu.CompilerParams(
            dimension_semantics=("parallel",)),
    )(node_features, edge_features, W_msg, w_e, b_msg.reshape(1, D),
      W_ih, W_hh, b_ih.reshape(1, 3 * D), b_hh.reshape(1, 3 * D),
      W_r1, b_r1.reshape(1, D), W_r2, b_r2.reshape(1, D),
      W_p, b_p.reshape(1, A))
    return out.reshape(B, A)


# speedup vs baseline: 1.0525x; 1.0206x over previous
import jax
import jax.numpy as jnp
from jax.experimental import pallas as pl
from jax.experimental.pallas import tpu as pltpu

_SCALE = 1.0507009873554805
_ALPHA = 1.6732632423543772
_DIAMETER = 2

_DNT = (((1,), (1,)), ((), ()))


def _selu(x):
    em1 = jnp.exp(jnp.minimum(x, 0.0)) - 1.0
    return _SCALE * jnp.where(x > 0, x, _ALPHA * em1)


def _mmt(x, w):
    return jax.lax.dot_general(x, w, _DNT,
                               preferred_element_type=jnp.float32)


def _mpnn_kernel(h_ref, ef_ref, Wmsg_ref, we_ref, bm_ref,
                 Wih_ref, Whh_ref, bih_ref, bhh_ref,
                 Wr1_ref, br1_ref, Wr2_ref, br2_ref, Wp_ref, bp_ref,
                 out_ref):
    h = h_ref[0]
    ef = ef_ref[0]
    w_e = we_ref[...]
    N = h.shape[0]
    D = h.shape[1]
    sa = _SCALE * _ALPHA
    Ws = Wmsg_ref[:, :D]
    Wt = Wmsg_ref[:, D:2 * D]
    bm = bm_ref[...][None, :]
    bih = bih_ref[...][None, :]
    bhh = bhh_ref[...][None, :]
    br1 = br1_ref[...][None, :]
    br2 = br2_ref[...][None, :]
    bp = bp_ref[...][None, :]

    ef_r = ef.astype(jnp.bfloat16).astype(jnp.float32)
    we_r = w_e.astype(jnp.bfloat16).astype(jnp.float32)
    E = ef_r[:, :, None] * we_r[0][None, None, :]

    for _ in range(_DIAMETER):
        Xs = _mmt(h, Ws)
        Xt = _mmt(h, Wt) + bm
        TS = 8
        pos = jnp.zeros((N, D), jnp.float32)
        esum = jnp.zeros((N, D), jnp.float32)
        for c in range(N // TS):
            sl = slice(c * TS, (c + 1) * TS)
            blk = Xs[sl][:, None, :] + Xt[None, :, :] + E[sl]
            pos = pos + jnp.sum(jnp.maximum(blk, 0.0), axis=0)
            esum = esum + jnp.sum(jnp.exp(jnp.minimum(blk, 0.0)), axis=0)
        agg = _SCALE * pos + sa * esum - (sa * N)
        gi = _mmt(agg, Wih_ref[...]) + bih
        gh = _mmt(h, Whh_ref[...]) + bhh
        i_r, i_z, i_n = gi[:, :D], gi[:, D:2 * D], gi[:, 2 * D:]
        h_r, h_z, h_n = gh[:, :D], gh[:, D:2 * D], gh[:, 2 * D:]
        r = jax.nn.sigmoid(i_r + h_r)
        z = jax.nn.sigmoid(i_z + h_z)
        n = jnp.tanh(i_n + r * h_n)
        h = (1.0 - z) * n + z * h

    ns = jnp.sum(h, axis=0, keepdims=True)
    r1 = _selu(_mmt(ns, Wr1_ref[...]) + br1)
    r2 = _selu(_mmt(r1, Wr2_ref[...]) + br2)
    b = pl.program_id(0)
    out_ref[pl.ds(b, 1), :] = _mmt(r2, Wp_ref[...]) + bp


def kernel(node_features, edge_features, adjacency_matrix,
           W_msg, b_msg, W_ih, W_hh, b_ih, b_hh,
           W_r1, b_r1, W_r2, b_r2, W_p, b_p):
    B, N, D = node_features.shape
    A = W_p.shape[0]

    w_e = W_msg[:, 2 * D].reshape(1, D)

    full = lambda shape: pl.BlockSpec(shape, lambda b: (0,) * len(shape))
    out = pl.pallas_call(
        _mpnn_kernel,
        grid=(B,),
        in_specs=[
            pl.BlockSpec((1, N, D), lambda b: (b, 0, 0)),
            pl.BlockSpec((1, N, N), lambda b: (b, 0, 0)),
            full((D, 2 * D + 1)), full((1, D)), full((D,)),
            full((3 * D, D)), full((3 * D, D)), full((3 * D,)),
            full((3 * D,)),
            full((D, D)), full((D,)), full((D, D)), full((D,)),
            full((A, D)), full((A,)),
        ],
        out_specs=pl.BlockSpec((B, A), lambda b: (0, 0)),
        out_shape=jax.ShapeDtypeStruct((B, A), jnp.float32),
    )(node_features, edge_features, W_msg, w_e, b_msg,
      W_ih, W_hh, b_ih, b_hh, W_r1, b_r1, W_r2, b_r2, W_p, b_p)
    return out


# speedup vs baseline: 1.0835x; 1.0295x over previous
import jax
import jax.numpy as jnp
from jax.experimental import pallas as pl
from jax.experimental.pallas import tpu as pltpu

_SCALE = 1.0507009873554805
_ALPHA = 1.6732632423543772
_DIAMETER = 2

_DNT = (((1,), (1,)), ((), ()))


def _selu(x):
    em1 = jnp.exp(jnp.minimum(x, 0.0)) - 1.0
    return _SCALE * jnp.where(x > 0, x, _ALPHA * em1)


def _mmt(x, w):
    return jax.lax.dot_general(x, w, _DNT,
                               preferred_element_type=jnp.float32)


def _mpnn_kernel(h_ref, ef_ref, Wmsg_ref, we_ref, bm_ref,
                 Wih_ref, Whh_ref, bih_ref, bhh_ref,
                 Wr1_ref, br1_ref, Wr2_ref, br2_ref, Wp_ref, bp_ref,
                 out_ref):
    h = h_ref[0]
    ef = ef_ref[0]
    w_e = we_ref[...]
    N = h.shape[0]
    D = h.shape[1]
    sa = _SCALE * _ALPHA
    Ws = Wmsg_ref[:, :D]
    Wt = Wmsg_ref[:, D:2 * D]
    bm = bm_ref[...][None, :]
    bih = bih_ref[...][None, :]
    bhh = bhh_ref[...][None, :]
    br1 = br1_ref[...][None, :]
    br2 = br2_ref[...][None, :]
    bp = bp_ref[...][None, :]

    ef_r = ef.astype(jnp.bfloat16).astype(jnp.float32)
    we_r = w_e.astype(jnp.bfloat16).astype(jnp.float32)
    E = ef_r[:, :, None] * we_r[0][None, None, :]
    SE = jnp.sum(E, axis=0)

    for _ in range(_DIAMETER):
        Xs = _mmt(h, Ws)
        Xt = _mmt(h, Wt) + bm
        TS = 8
        mns = jnp.zeros((N, D), jnp.float32)
        esum = jnp.zeros((N, D), jnp.float32)
        for c in range(N // TS):
            sl = slice(c * TS, (c + 1) * TS)
            blk = Xs[sl][:, None, :] + Xt[None, :, :] + E[sl]
            mn = jnp.minimum(blk, 0.0)
            mns = mns + jnp.sum(mn, axis=0)
            esum = esum + jnp.sum(jnp.exp(mn), axis=0)
        tot = jnp.sum(Xs, axis=0, keepdims=True) + N * Xt + SE
        agg = _SCALE * (tot - mns) + sa * esum - (sa * N)
        gi = _mmt(agg, Wih_ref[...]) + bih
        gh = _mmt(h, Whh_ref[...]) + bhh
        i_r, i_z, i_n = gi[:, :D], gi[:, D:2 * D], gi[:, 2 * D:]
        h_r, h_z, h_n = gh[:, :D], gh[:, D:2 * D], gh[:, 2 * D:]
        r = jax.nn.sigmoid(i_r + h_r)
        z = jax.nn.sigmoid(i_z + h_z)
        n = jnp.tanh(i_n + r * h_n)
        h = (1.0 - z) * n + z * h

    ns = jnp.sum(h, axis=0, keepdims=True)
    r1 = _selu(_mmt(ns, Wr1_ref[...]) + br1)
    r2 = _selu(_mmt(r1, Wr2_ref[...]) + br2)
    b = pl.program_id(0)
    out_ref[pl.ds(b, 1), :] = _mmt(r2, Wp_ref[...]) + bp


def kernel(node_features, edge_features, adjacency_matrix,
           W_msg, b_msg, W_ih, W_hh, b_ih, b_hh,
           W_r1, b_r1, W_r2, b_r2, W_p, b_p):
    B, N, D = node_features.shape
    A = W_p.shape[0]

    w_e = W_msg[:, 2 * D].reshape(1, D)

    full = lambda shape: pl.BlockSpec(shape, lambda b: (0,) * len(shape))
    out = pl.pallas_call(
        _mpnn_kernel,
        grid=(B,),
        in_specs=[
            pl.BlockSpec((1, N, D), lambda b: (b, 0, 0)),
            pl.BlockSpec((1, N, N), lambda b: (b, 0, 0)),
            full((D, 2 * D + 1)), full((1, D)), full((D,)),
            full((3 * D, D)), full((3 * D, D)), full((3 * D,)),
            full((3 * D,)),
            full((D, D)), full((D,)), full((D, D)), full((D,)),
            full((A, D)), full((A,)),
        ],
        out_specs=pl.BlockSpec((B, A), lambda b: (0, 0)),
        out_shape=jax.ShapeDtypeStruct((B, A), jnp.float32),
    )(node_features, edge_features, W_msg, w_e, b_msg,
      W_ih, W_hh, b_ih, b_hh, W_r1, b_r1, W_r2, b_r2, W_p, b_p)
    return out
